# Initial kernel scaffold; baseline (speedup 1.0000x reference)
import jax
import jax.numpy as jnp
from jax.experimental import pallas as pl


def kernel(In, NNsites, Weights, bias):
    raise NotImplementedError("write your pallas kernel here")


# speedup vs baseline: 11.9065x; 11.9065x over previous
import functools

import jax
import jax.numpy as jnp
from jax import lax
from jax.experimental import pallas as pl
from jax.experimental.pallas import tpu as pltpu
from jax.experimental.pallas import tpu_sc as plsc


_TS_A = 2048
_TS_B = 2048


def _stage_a_body(x_ref, wt_ref, p1_ref, p2_ref):
    wsum = jnp.sum(wt_ref[...], axis=1, keepdims=True)
    n = wsum.shape[0] // 2
    w1 = wsum[0:n, :].reshape(1, n, 1)
    w2 = wsum[n:, :].reshape(1, n, 1)
    x = x_ref[...]
    p1_ref[...] = jnp.sum(x * w1, axis=1)
    p2_ref[...] = jnp.sum(x * w2, axis=1)


def _stage_a(In, wt):
    B, NSpec, S = In.shape
    nt = pl.cdiv(S, _TS_A)
    return pl.pallas_call(
        _stage_a_body,
        grid=(nt,),
        in_specs=[
            pl.BlockSpec((B, NSpec, _TS_A), lambda i: (0, 0, i)),
            pl.BlockSpec(wt.shape, lambda i: (0, 0)),
        ],
        out_specs=[
            pl.BlockSpec((B, _TS_A), lambda i: (0, i)),
            pl.BlockSpec((B, _TS_A), lambda i: (0, i)),
        ],
        out_shape=[
            jax.ShapeDtypeStruct((B, S), jnp.float32),
            jax.ShapeDtypeStruct((B, S), jnp.float32),
        ],
    )(In, wt)


def _sc_gather(p2, nn):
    B, S = p2.shape
    Z = nn.shape[0]
    info = plsc.get_sparse_core_info()
    nw = info.num_cores * info.num_subcores
    per = (Z * B) // nw
    mesh = plsc.VectorSubcoreMesh(core_axis_name="c", subcore_axis_name="s")

    @functools.partial(
        pl.kernel,
        mesh=mesh,
        out_type=jax.ShapeDtypeStruct((Z, B, S), jnp.float32),
        compiler_params=pltpu.CompilerParams(needs_layout_passes=False),
        scratch_types=[
            pltpu.VMEM((S,), jnp.float32),
            pltpu.VMEM((S,), jnp.int32),
            pltpu.VMEM((S,), jnp.float32),
        ],
    )
    def k(p2_hbm, nn_hbm, g_hbm, p2_v, idx_v, out_v):
        wid = lax.axis_index("s") * info.num_cores + lax.axis_index("c")
        b = wid % B
        zg = wid // B
        pltpu.sync_copy(p2_hbm.at[b], p2_v)
        for j in range(per):
            z = zg * per + j
            pltpu.sync_copy(nn_hbm.at[z], idx_v)

            def body(i, carry):
                sl = pl.ds(i * 16, 16)
                out_v[sl] = plsc.load_gather(p2_v, [idx_v[sl]])
                return carry

            lax.fori_loop(0, S // 16, body, 0)
            pltpu.sync_copy(out_v, g_hbm.at[z, b])

    return k(p2, nn)


def _stage_b_body(p1_ref, g_ref, bt_ref, out_ref):
    z = pl.program_id(1)
    bs = jnp.sum(bt_ref[...], axis=1, keepdims=True)
    bs = bs.reshape(1, bs.shape[0], 1)
    c = p1_ref[...] + g_ref[0]
    x = c[:, None, :] + bs
    v = jnp.maximum(x, 0.0) + jnp.log1p(jnp.exp(-jnp.abs(x)))

    @pl.when(z == 0)
    def _():
        out_ref[...] = v

    @pl.when(z > 0)
    def _():
        out_ref[...] = out_ref[...] + v


def _stage_b(p1, g, bt):
    Z, B, S = g.shape
    NSpec = bt.shape[0]
    nt = pl.cdiv(S, _TS_B)
    return pl.pallas_call(
        _stage_b_body,
        grid=(nt, Z),
        in_specs=[
            pl.BlockSpec((B, _TS_B), lambda t, z: (0, t)),
            pl.BlockSpec((1, B, _TS_B), lambda t, z: (z, 0, t)),
            pl.BlockSpec(bt.shape, lambda t, z: (0, 0)),
        ],
        out_specs=pl.BlockSpec((B, NSpec, _TS_B), lambda t, z: (0, 0, t)),
        out_shape=jax.ShapeDtypeStruct((B, NSpec, S), jnp.float32),
    )(p1, g, bt)


def kernel(In, NNsites, Weights, bias):
    wt = Weights[:, 0, :].T
    bt = bias.T
    nn = NNsites[1:]
    p1, p2 = _stage_a(In, wt)
    g = _sc_gather(p2, nn)
    return _stage_b(p1, g, bt)
